# baseline (speedup 1.0000x reference)
import numpy as np
import jax
import jax.numpy as jnp
from jax.experimental import pallas as pl
from jax.experimental.pallas import tpu as pltpu

TOPK = 64
E = 64
CBLK = 128


def _ce(p, i, j, desc):
    a, b = p[i], p[j]
    hi = jnp.maximum(a, b)
    lo = jnp.minimum(a, b)
    p[i], p[j] = (hi, lo) if desc else (lo, hi)


def _oe_pairs(n):
    pairs = []
    pp = 1
    while pp < n:
        k = pp
        while k >= 1:
            for j in range(k % pp, n - k, 2 * k):
                for i in range(0, min(k, n - j - k)):
                    if (i + j) // (2 * pp) == (i + j + k) // (2 * pp):
                        pairs.append((i + j, i + j + k))
            k //= 2
        pp *= 2
    return pairs


_OE64 = _oe_pairs(E)


def _sort_pieces(p, desc):
    for i, j in _OE64:
        _ce(p, i, j, desc)


def _cleanup(p, desc):
    j = E // 2
    while j >= 1:
        for i in range(E):
            if i & j == 0:
                _ce(p, i, i + j, desc)
        j //= 2


def _topk_kernel(x_ref, o_ref):
    g = x_ref.shape[1] // E
    gh = g // 2
    pl_ = [x_ref[0, i * g : i * g + gh, :] for i in range(E)]
    pr_ = [x_ref[0, i * g + gh : (i + 1) * g, :] for i in range(E)]
    _sort_pieces(pl_, True)
    _sort_pieces(pr_, False)
    g = gh
    while True:
        y = [jnp.maximum(a, b) for a, b in zip(pl_, pr_)]
        if g == 1:
            _cleanup(y, True)
            for i in range(E):
                o_ref[0, i, :] = y[i][0, :]
            return
        pl_ = [t[: g // 2] for t in y]
        pr_ = [t[g // 2 :] for t in y]
        _cleanup(pl_, True)
        _cleanup(pr_, False)
        g //= 2


def kernel(inputs):
    b, l, c = inputs.shape
    return pl.pallas_call(
        _topk_kernel,
        grid=(b, c // CBLK),
        in_specs=[pl.BlockSpec((1, l, CBLK), lambda i, j: (i, 0, j))],
        out_specs=pl.BlockSpec((1, TOPK, CBLK), lambda i, j: (i, 0, j)),
        out_shape=jax.ShapeDtypeStruct((b, TOPK, c), inputs.dtype),
        compiler_params=pltpu.CompilerParams(
            dimension_semantics=("parallel", "parallel")
        ),
    )(inputs)


# speedup vs baseline: 1.2425x; 1.2425x over previous
import numpy as np
import jax
import jax.numpy as jnp
from jax.experimental import pallas as pl
from jax.experimental.pallas import tpu as pltpu

TOPK = 64
E = 64
CBLK = 128


def _ce(p, i, j, desc):
    a, b = p[i], p[j]
    hi = jnp.maximum(a, b)
    lo = jnp.minimum(a, b)
    p[i], p[j] = (hi, lo) if desc else (lo, hi)


def _oe_pairs(n):
    pairs = []
    pp = 1
    while pp < n:
        k = pp
        while k >= 1:
            for j in range(k % pp, n - k, 2 * k):
                for i in range(0, min(k, n - j - k)):
                    if (i + j) // (2 * pp) == (i + j + k) // (2 * pp):
                        pairs.append((i + j, i + j + k))
            k //= 2
        pp *= 2
    return pairs


_OE64 = _oe_pairs(E)


def _sort_pieces(p, desc):
    for i, j in _OE64:
        _ce(p, i, j, desc)


def _cleanup(p, desc):
    j = E // 2
    while j >= 1:
        for i in range(E):
            if i & j == 0:
                _ce(p, i, i + j, desc)
        j //= 2


CB = 16


def _topk_kernel(x_ref, o_ref):
    g = x_ref.shape[1] // E
    nb = g // CB
    batches = []
    for bi in range(nb):
        off = bi * CB
        batches.append(
            [x_ref[0, i * g + off : i * g + off + CB, :] for i in range(E)]
        )
    for bi in range(nb):
        _sort_pieces(batches[bi], bi < nb // 2)
    while len(batches) > 1:
        h = len(batches) // 2
        batches = [
            [jnp.maximum(a, b) for a, b in zip(batches[bi], batches[bi + h])]
            for bi in range(h)
        ]
        if h == 1:
            break
        for bi in range(h):
            _cleanup(batches[bi], bi < h // 2)
    y = batches[0]
    g = CB
    while True:
        pl_ = [t[: g // 2] for t in y]
        pr_ = [t[g // 2 :] for t in y]
        _cleanup(pl_, True)
        _cleanup(pr_, False)
        y = [jnp.maximum(a, b) for a, b in zip(pl_, pr_)]
        g //= 2
        if g == 1:
            _cleanup(y, True)
            for i in range(E):
                o_ref[0, i, :] = y[i][0, :]
            return


def kernel(inputs):
    b, l, c = inputs.shape
    return pl.pallas_call(
        _topk_kernel,
        grid=(b, c // CBLK),
        in_specs=[pl.BlockSpec((1, l, CBLK), lambda i, j: (i, 0, j))],
        out_specs=pl.BlockSpec((1, TOPK, CBLK), lambda i, j: (i, 0, j)),
        out_shape=jax.ShapeDtypeStruct((b, TOPK, c), inputs.dtype),
        compiler_params=pltpu.CompilerParams(
            dimension_semantics=("parallel", "parallel")
        ),
    )(inputs)


# speedup vs baseline: 1.4282x; 1.1494x over previous
import numpy as np
import jax
import jax.numpy as jnp
from jax.experimental import pallas as pl
from jax.experimental.pallas import tpu as pltpu

TOPK = 64
E = 64
CBLK = 128


def _ce(p, i, j, desc):
    a, b = p[i], p[j]
    hi = jnp.maximum(a, b)
    lo = jnp.minimum(a, b)
    p[i], p[j] = (hi, lo) if desc else (lo, hi)


def _oe_pairs(n):
    pairs = []
    pp = 1
    while pp < n:
        k = pp
        while k >= 1:
            for j in range(k % pp, n - k, 2 * k):
                for i in range(0, min(k, n - j - k)):
                    if (i + j) // (2 * pp) == (i + j + k) // (2 * pp):
                        pairs.append((i + j, i + j + k))
            k //= 2
        pp *= 2
    return pairs


_OE64 = _oe_pairs(E)


def _sort_pieces(p, desc):
    for i, j in _OE64:
        _ce(p, i, j, desc)


def _cleanup(p, desc):
    j = E // 2
    while j >= 1:
        for i in range(E):
            if i & j == 0:
                _ce(p, i, i + j, desc)
        j //= 2


CB = 8


def _topk_kernel(x_ref, o_ref):
    g = x_ref.shape[1] // E
    nb = g // CB
    batches = []
    for bi in range(nb):
        off = bi * CB
        batches.append(
            [x_ref[0, i * g + off : i * g + off + CB, :] for i in range(E)]
        )
    for bi in range(nb):
        _sort_pieces(batches[bi], bi < nb // 2)
    while len(batches) > 1:
        h = len(batches) // 2
        batches = [
            [jnp.maximum(a, b) for a, b in zip(batches[bi], batches[bi + h])]
            for bi in range(h)
        ]
        if h == 1:
            break
        for bi in range(h):
            _cleanup(batches[bi], bi < h // 2)
    y = batches[0]
    g = CB
    while True:
        pl_ = [t[: g // 2] for t in y]
        pr_ = [t[g // 2 :] for t in y]
        _cleanup(pl_, True)
        _cleanup(pr_, False)
        y = [jnp.maximum(a, b) for a, b in zip(pl_, pr_)]
        g //= 2
        if g == 1:
            _cleanup(y, True)
            for i in range(E):
                o_ref[0, i, :] = y[i][0, :]
            return


def kernel(inputs):
    b, l, c = inputs.shape
    return pl.pallas_call(
        _topk_kernel,
        grid=(b, c // CBLK),
        in_specs=[pl.BlockSpec((1, l, CBLK), lambda i, j: (i, 0, j))],
        out_specs=pl.BlockSpec((1, TOPK, CBLK), lambda i, j: (i, 0, j)),
        out_shape=jax.ShapeDtypeStruct((b, TOPK, c), inputs.dtype),
        compiler_params=pltpu.CompilerParams(
            dimension_semantics=("parallel", "parallel")
        ),
    )(inputs)
